# Initial kernel scaffold; baseline (speedup 1.0000x reference)
import jax
import jax.numpy as jnp
from jax.experimental import pallas as pl


def kernel(x, edge_index, W1, b1, W2, b2):
    raise NotImplementedError("write your pallas kernel here")


# speedup vs baseline: 9.8274x; 9.8274x over previous
import functools

import jax
import jax.numpy as jnp
from jax import lax
from jax.experimental import pallas as pl
from jax.experimental.pallas import tpu as pltpu
from jax.experimental.pallas import tpu_sc as plsc

N_NODES = 10000
N_PAD = 10240
D_IN = 128
D_HID = 128
D_OUT = 64

NC = 2
NS = 16
L = 16
NW = NC * NS
CHUNK = 128
E_PAD = 327680
CHUNKS = E_PAD // (NW * CHUNK)
ROWS_PER_TILE = N_PAD // NS

BM = 256


def _mesh():
    return plsc.VectorSubcoreMesh(core_axis_name="c", subcore_axis_name="s")


def _make_deg_kernel():
    @functools.partial(
        pl.kernel,
        mesh=_mesh(),
        out_type=[jax.ShapeDtypeStruct((N_PAD, L), jnp.float32),
                  jax.ShapeDtypeStruct((N_PAD, L), jnp.float32)],
        scratch_types=[
            pltpu.VMEM((CHUNKS, CHUNK), jnp.int32),
            pltpu.VMEM((CHUNK, L), jnp.float32),
            pltpu.VMEM_SHARED((N_PAD, L), jnp.float32),
        ],
        compiler_params=pltpu.CompilerParams(use_tc_tiling_on_sc=False),
    )
    def deg_kernel(dst_hbm, ones_hbm, zeros_hbm, out_a, out_b,
                   dst_v, ones_v, acc):
        c = lax.axis_index("c")
        s = lax.axis_index("s")
        wid = c * NS + s
        r0 = s * ROWS_PER_TILE
        stripe = pl.ds(r0, ROWS_PER_TILE)
        pltpu.sync_copy(zeros_hbm.at[stripe], acc.at[stripe])
        pltpu.sync_copy(ones_hbm, ones_v)
        pltpu.sync_copy(dst_hbm.at[pl.ds(wid * CHUNKS, CHUNKS)], dst_v)
        plsc.subcore_barrier()

        def body(j, carry):
            pltpu.sync_copy(ones_v, acc.at[dst_v.at[j]], add=True)
            return carry

        lax.fori_loop(0, CHUNKS, body, 0)
        plsc.subcore_barrier()

        @pl.when(c == 0)
        def _():
            pltpu.sync_copy(acc.at[stripe], out_a.at[stripe])

        @pl.when(c == 1)
        def _():
            pltpu.sync_copy(acc.at[stripe], out_b.at[stripe])

    return deg_kernel


def _make_agg_kernel(d_feat):
    @functools.partial(
        pl.kernel,
        mesh=_mesh(),
        out_type=[jax.ShapeDtypeStruct((N_PAD, d_feat), jnp.float32),
                  jax.ShapeDtypeStruct((N_PAD, d_feat), jnp.float32)],
        scratch_types=[
            pltpu.VMEM((CHUNKS, CHUNK), jnp.int32),
            pltpu.VMEM((CHUNKS, CHUNK), jnp.int32),
            pltpu.VMEM((CHUNK, d_feat), jnp.float32),
            pltpu.VMEM_SHARED((N_PAD, d_feat), jnp.float32),
            pltpu.SemaphoreType.DMA,
        ],
        compiler_params=pltpu.CompilerParams(use_tc_tiling_on_sc=False),
    )
    def agg_kernel(xs_hbm, src_hbm, dst_hbm, out_a, out_b,
                   src_v, dst_v, rows_v, acc, sem):
        c = lax.axis_index("c")
        s = lax.axis_index("s")
        wid = c * NS + s
        r0 = s * ROWS_PER_TILE
        stripe = pl.ds(r0, ROWS_PER_TILE)
        pltpu.sync_copy(xs_hbm.at[stripe], acc.at[stripe])
        pltpu.sync_copy(src_hbm.at[pl.ds(wid * CHUNKS, CHUNKS)], src_v)
        pltpu.sync_copy(dst_hbm.at[pl.ds(wid * CHUNKS, CHUNKS)], dst_v)
        plsc.subcore_barrier()

        def body(j, carry):
            pltpu.async_copy(xs_hbm.at[src_v.at[j]], rows_v, sem).wait()
            pltpu.sync_copy(rows_v, acc.at[dst_v.at[j]], add=True)
            return carry

        lax.fori_loop(0, CHUNKS, body, 0)
        plsc.subcore_barrier()

        @pl.when(c == 0)
        def _():
            pltpu.sync_copy(acc.at[stripe], out_a.at[stripe])

        @pl.when(c == 1)
        def _():
            pltpu.sync_copy(acc.at[stripe], out_b.at[stripe])

    return agg_kernel


_deg_call = _make_deg_kernel()
_agg_hid = _make_agg_kernel(D_HID)
_agg_out = _make_agg_kernel(D_OUT)


def _dis(ca_ref, cb_ref):
    return lax.rsqrt(ca_ref[:, :1] + cb_ref[:, :1] + 1.0)


def _xs1_body(x_ref, w_ref, ca_ref, cb_ref, o_ref):
    dis = _dis(ca_ref, cb_ref)
    o_ref[...] = jnp.dot(x_ref[...], w_ref[...],
                         preferred_element_type=jnp.float32) * dis


def _tc_xs1(x, w1, cnt_a, cnt_b):
    return pl.pallas_call(
        _xs1_body,
        grid=(N_PAD // BM,),
        in_specs=[
            pl.BlockSpec((BM, D_IN), lambda i: (i, 0)),
            pl.BlockSpec((D_IN, D_HID), lambda i: (0, 0)),
            pl.BlockSpec((BM, L), lambda i: (i, 0)),
            pl.BlockSpec((BM, L), lambda i: (i, 0)),
        ],
        out_specs=pl.BlockSpec((BM, D_HID), lambda i: (i, 0)),
        out_shape=jax.ShapeDtypeStruct((N_PAD, D_HID), jnp.float32),
    )(x, w1, cnt_a, cnt_b)


def _xs2_body(aa_ref, ab_ref, xs_ref, ca_ref, cb_ref, b1_ref, w2_ref, o_ref):
    dis = _dis(ca_ref, cb_ref)
    h = dis * (aa_ref[...] + ab_ref[...] - xs_ref[...]) + b1_ref[...]
    h = jnp.maximum(h, 0.0)
    o_ref[...] = jnp.dot(h, w2_ref[...],
                         preferred_element_type=jnp.float32) * dis


def _tc_xs2(agg_a, agg_b, xs1, cnt_a, cnt_b, b1, w2):
    return pl.pallas_call(
        _xs2_body,
        grid=(N_PAD // BM,),
        in_specs=[
            pl.BlockSpec((BM, D_HID), lambda i: (i, 0)),
            pl.BlockSpec((BM, D_HID), lambda i: (i, 0)),
            pl.BlockSpec((BM, D_HID), lambda i: (i, 0)),
            pl.BlockSpec((BM, L), lambda i: (i, 0)),
            pl.BlockSpec((BM, L), lambda i: (i, 0)),
            pl.BlockSpec((1, D_HID), lambda i: (0, 0)),
            pl.BlockSpec((D_HID, D_OUT), lambda i: (0, 0)),
        ],
        out_specs=pl.BlockSpec((BM, D_OUT), lambda i: (i, 0)),
        out_shape=jax.ShapeDtypeStruct((N_PAD, D_OUT), jnp.float32),
    )(agg_a, agg_b, xs1, cnt_a, cnt_b, b1, w2)


def _final_body(aa_ref, ab_ref, xs_ref, ca_ref, cb_ref, b2_ref, o_ref):
    dis = _dis(ca_ref, cb_ref)
    v = dis * (aa_ref[...] + ab_ref[...] - xs_ref[...]) + b2_ref[...]
    m = jnp.max(v, axis=1, keepdims=True)
    e = jnp.exp(v - m)
    o_ref[...] = (v - m) - jnp.log(jnp.sum(e, axis=1, keepdims=True))


def _tc_final(agg_a, agg_b, xs2, cnt_a, cnt_b, b2):
    return pl.pallas_call(
        _final_body,
        grid=(N_PAD // BM,),
        in_specs=[
            pl.BlockSpec((BM, D_OUT), lambda i: (i, 0)),
            pl.BlockSpec((BM, D_OUT), lambda i: (i, 0)),
            pl.BlockSpec((BM, D_OUT), lambda i: (i, 0)),
            pl.BlockSpec((BM, L), lambda i: (i, 0)),
            pl.BlockSpec((BM, L), lambda i: (i, 0)),
            pl.BlockSpec((1, D_OUT), lambda i: (0, 0)),
        ],
        out_specs=pl.BlockSpec((BM, D_OUT), lambda i: (i, 0)),
        out_shape=jax.ShapeDtypeStruct((N_PAD, D_OUT), jnp.float32),
    )(agg_a, agg_b, xs2, cnt_a, cnt_b, b2)


def kernel(x, edge_index, W1, b1, W2, b2):
    n_edges = edge_index.shape[1]
    src = edge_index[0].astype(jnp.int32)
    dst = edge_index[1].astype(jnp.int32)
    pad = jnp.full((E_PAD - n_edges,), N_NODES, dtype=jnp.int32)
    src2d = jnp.concatenate([src, pad]).reshape(NW * CHUNKS, CHUNK)
    dst2d = jnp.concatenate([dst, pad]).reshape(NW * CHUNKS, CHUNK)
    xp = jnp.pad(x.astype(jnp.float32), ((0, N_PAD - N_NODES), (0, 0)))
    ones = jnp.ones((CHUNK, L), jnp.float32)
    zeros = jnp.zeros((N_PAD, L), jnp.float32)

    cnt_a, cnt_b = _deg_call(dst2d, ones, zeros)

    xs1 = _tc_xs1(xp, W1.astype(jnp.float32), cnt_a, cnt_b)
    agg_a, agg_b = _agg_hid(xs1, src2d, dst2d)
    xs2 = _tc_xs2(agg_a, agg_b, xs1, cnt_a, cnt_b,
                  b1.astype(jnp.float32).reshape(1, D_HID),
                  W2.astype(jnp.float32))
    agg2_a, agg2_b = _agg_out(xs2, src2d, dst2d)
    out = _tc_final(agg2_a, agg2_b, xs2, cnt_a, cnt_b,
                    b2.astype(jnp.float32).reshape(1, D_OUT))
    return out[:N_NODES]
